# baseline (device time: 59324 ns/iter reference)
import jax
import jax.numpy as jnp
from jax import lax
from jax.experimental import pallas as pl
from jax.experimental.pallas import tpu as pltpu


def kernel(x, k, Wp):
    b, hh, ww, c = x.shape
    n_global = (2 * hh) * (2 * ww)
    eps = 1e-5

    def body(
        x_ref, k_ref, wp_ref, out_ref,
        pad_ref,
        row_send, col_send, cor_send, stat_send,
        row_recv, col_recv, cor_recv, stat_recv,
        send_sems, recv_sems, stat_sems,
    ):
        my_x = lax.axis_index("x")
        my_y = lax.axis_index("y")
        nbx = 1 - my_x
        nby = 1 - my_y
        my_id = my_x * 2 + my_y

        barrier = pltpu.get_barrier_semaphore()
        for dev in ((nbx, my_y), (my_x, nby), (nbx, nby)):
            pl.semaphore_signal(
                barrier, inc=1, device_id=dev,
                device_id_type=pl.DeviceIdType.MESH,
            )
        pl.semaphore_wait(barrier, 3)

        send_row = (1 - my_x) * (hh - 1)
        row_send[...] = x_ref[:, pl.ds(send_row, 1), :, :]

        @pl.when(my_y == 0)
        def _():
            col_send[...] = x_ref[:, :, :, ww - 1]
            cor_send[...] = x_ref[:, pl.ds(send_row, 1), :, ww - 1]

        @pl.when(my_y == 1)
        def _():
            col_send[...] = x_ref[:, :, :, 0]
            cor_send[...] = x_ref[:, pl.ds(send_row, 1), :, 0]

        rdma_row = pltpu.make_async_remote_copy(
            src_ref=row_send, dst_ref=row_recv,
            send_sem=send_sems.at[0], recv_sem=recv_sems.at[0],
            device_id=(nbx, my_y), device_id_type=pl.DeviceIdType.MESH,
        )
        rdma_col = pltpu.make_async_remote_copy(
            src_ref=col_send, dst_ref=col_recv,
            send_sem=send_sems.at[1], recv_sem=recv_sems.at[1],
            device_id=(my_x, nby), device_id_type=pl.DeviceIdType.MESH,
        )
        rdma_cor = pltpu.make_async_remote_copy(
            src_ref=cor_send, dst_ref=cor_recv,
            send_sem=send_sems.at[2], recv_sem=recv_sems.at[2],
            device_id=(nbx, nby), device_id_type=pl.DeviceIdType.MESH,
        )
        rdma_row.start()
        rdma_col.start()
        rdma_cor.start()

        for bi in range(b):
            xb = x_ref[bi]
            stat_send[0, bi] = jnp.sum(xb, axis=(0, 2))
            stat_send[1, bi] = jnp.sum(xb * xb, axis=(0, 2))
        stat_recv[pl.ds(my_id, 1)] = stat_send[...].reshape(1, 2, b, c)

        stat_rdmas = []
        for i, dev in enumerate(((nbx, my_y), (my_x, nby), (nbx, nby))):
            r = pltpu.make_async_remote_copy(
                src_ref=stat_send,
                dst_ref=stat_recv.at[my_id],
                send_sem=send_sems.at[3 + i],
                recv_sem=stat_sems.at[my_id],
                device_id=dev, device_id_type=pl.DeviceIdType.MESH,
            )
            r.start()
            stat_rdmas.append(r)

        for pid in (nbx * 2 + my_y, my_x * 2 + nby, nbx * 2 + nby):
            rr = pltpu.make_async_remote_copy(
                src_ref=stat_send,
                dst_ref=stat_recv.at[pid],
                send_sem=send_sems.at[3],
                recv_sem=stat_sems.at[pid],
                device_id=(my_x, my_y), device_id_type=pl.DeviceIdType.MESH,
            )
            rr.wait_recv()

        tot = (
            stat_recv[0] + stat_recv[1] + stat_recv[2] + stat_recv[3]
        )
        mean = tot[0] / n_global
        var = tot[1] / n_global - mean * mean
        inv = lax.rsqrt(var + eps)

        nh = (1 - my_x) * (hh + 1)
        eh = my_x * (hh + 1)
        eh_adj = my_x * (hh - 1) + 1

        pad_ref[:, 1:hh + 1, :, 1:ww + 1] = x_ref[...]
        rdma_row.wait_recv()
        pad_ref[:, pl.ds(nh, 1), :, 1:ww + 1] = row_recv[...]
        rdma_col.wait_recv()
        rdma_cor.wait_recv()

        @pl.when(my_y == 0)
        def _():
            pad_ref[:, 1:hh + 1, :, ww + 1] = col_recv[...]
            pad_ref[:, pl.ds(nh, 1), :, ww + 1] = cor_recv[...]
            pad_ref[:, :, :, 0] = pad_ref[:, :, :, 1]

        @pl.when(my_y == 1)
        def _():
            pad_ref[:, 1:hh + 1, :, 0] = col_recv[...]
            pad_ref[:, pl.ds(nh, 1), :, 0] = cor_recv[...]
            pad_ref[:, :, :, ww + 1] = pad_ref[:, :, :, ww]

        pad_ref[:, pl.ds(eh, 1), :, :] = pad_ref[:, pl.ds(eh_adj, 1), :, :]

        kk = k_ref[...]
        ksum = jnp.sum(kk, axis=(0, 1))
        CH = 32
        wpb = jnp.broadcast_to(wp_ref[...], (CH, c, c))
        for bi in range(b):
            for h0 in range(0, hh, CH):
                conv = jnp.zeros((CH, c, ww), jnp.float32)
                for di in range(3):
                    for dj in range(3):
                        conv = conv + (
                            pad_ref[bi, h0 + di:h0 + di + CH, :, dj:dj + ww]
                            * kk[di, dj][:, None]
                        )
                iv = inv[bi].reshape(1, c, 1)
                off = (inv[bi] * mean[bi] * ksum).reshape(1, c, 1)
                conv = conv * iv - off
                a = conv * jax.nn.sigmoid(conv)
                y = lax.dot_general(
                    wpb, a, (((1,), (1,)), ((0,), (0,))),
                    preferred_element_type=jnp.float32,
                )
                out_ref[bi, h0:h0 + CH] = x_ref[bi, h0:h0 + CH] + y

        rdma_row.wait_send()
        rdma_col.wait_send()
        rdma_cor.wait_send()
        for r in stat_rdmas:
            r.wait_send()

    xt = jnp.transpose(x, (0, 1, 3, 2))
    out_t = pl.pallas_call(
        body,
        out_shape=jax.ShapeDtypeStruct((b, hh, c, ww), jnp.float32),
        in_specs=[
            pl.BlockSpec(memory_space=pltpu.VMEM),
            pl.BlockSpec(memory_space=pltpu.VMEM),
            pl.BlockSpec(memory_space=pltpu.VMEM),
        ],
        out_specs=pl.BlockSpec(memory_space=pltpu.VMEM),
        scratch_shapes=[
            pltpu.VMEM((b, hh + 2, c, ww + 2), jnp.float32),
            pltpu.VMEM((b, 1, c, ww), jnp.float32),
            pltpu.VMEM((b, hh, c), jnp.float32),
            pltpu.VMEM((b, 1, c), jnp.float32),
            pltpu.VMEM((2, b, c), jnp.float32),
            pltpu.VMEM((b, 1, c, ww), jnp.float32),
            pltpu.VMEM((b, hh, c), jnp.float32),
            pltpu.VMEM((b, 1, c), jnp.float32),
            pltpu.VMEM((4, 2, b, c), jnp.float32),
            pltpu.SemaphoreType.DMA((6,)),
            pltpu.SemaphoreType.DMA((3,)),
            pltpu.SemaphoreType.DMA((4,)),
        ],
        compiler_params=pltpu.CompilerParams(
            collective_id=0, vmem_limit_bytes=120 * 1024 * 1024
        ),
    )(xt, k, Wp)
    return jnp.transpose(out_t, (0, 1, 3, 2))


# device time: 37227 ns/iter; 1.5936x vs baseline; 1.5936x over previous
import jax
import jax.numpy as jnp
from jax import lax
from jax.experimental import pallas as pl
from jax.experimental.pallas import tpu as pltpu


def kernel(x, k, Wp):
    b, hh, ww, c = x.shape
    n_global = (2 * hh) * (2 * ww)
    eps = 1e-5

    def body(
        x_ref, k_ref, wp_ref, out_ref,
        row_send, col_send, cor_send, stat_send,
        row_recv, col_recv, cor_recv, stat_recv,
        send_sems, recv_sems, stat_sems,
    ):
        my_x = lax.axis_index("x")
        my_y = lax.axis_index("y")
        nbx = 1 - my_x
        nby = 1 - my_y
        my_id = my_x * 2 + my_y

        barrier = pltpu.get_barrier_semaphore()
        for dev in ((nbx, my_y), (my_x, nby), (nbx, nby)):
            pl.semaphore_signal(
                barrier, inc=1, device_id=dev,
                device_id_type=pl.DeviceIdType.MESH,
            )
        pl.semaphore_wait(barrier, 3)

        send_row = (1 - my_x) * (hh - 1)
        row_send[...] = x_ref[:, pl.ds(send_row, 1), :, :]

        own_l = x_ref[:, :, :, 0]
        own_r = x_ref[:, :, :, ww - 1]
        col_send[...] = jnp.where(my_y == 0, own_r, own_l)
        cor_send[...] = col_send[:, pl.ds(send_row, 1), :]

        rdma_row = pltpu.make_async_remote_copy(
            src_ref=row_send, dst_ref=row_recv,
            send_sem=send_sems.at[0], recv_sem=recv_sems.at[0],
            device_id=(nbx, my_y), device_id_type=pl.DeviceIdType.MESH,
        )
        rdma_col = pltpu.make_async_remote_copy(
            src_ref=col_send, dst_ref=col_recv,
            send_sem=send_sems.at[1], recv_sem=recv_sems.at[1],
            device_id=(my_x, nby), device_id_type=pl.DeviceIdType.MESH,
        )
        rdma_cor = pltpu.make_async_remote_copy(
            src_ref=cor_send, dst_ref=cor_recv,
            send_sem=send_sems.at[2], recv_sem=recv_sems.at[2],
            device_id=(nbx, nby), device_id_type=pl.DeviceIdType.MESH,
        )
        rdma_row.start()
        rdma_col.start()
        rdma_cor.start()

        for bi in range(b):
            xb = x_ref[bi]
            stat_send[0, bi] = jnp.sum(xb, axis=(0, 2))
            stat_send[1, bi] = jnp.sum(xb * xb, axis=(0, 2))
        stat_recv[pl.ds(my_id, 1)] = stat_send[...].reshape(1, 2, b, c)

        stat_rdmas = []
        for i, dev in enumerate(((nbx, my_y), (my_x, nby), (nbx, nby))):
            r = pltpu.make_async_remote_copy(
                src_ref=stat_send,
                dst_ref=stat_recv.at[my_id],
                send_sem=send_sems.at[3 + i],
                recv_sem=stat_sems.at[my_id],
                device_id=dev, device_id_type=pl.DeviceIdType.MESH,
            )
            r.start()
            stat_rdmas.append(r)

        rdma_row.wait_recv()
        rdma_col.wait_recv()
        rdma_cor.wait_recv()

        toprow = jnp.where(my_x == 0, x_ref[:, 0:1], row_recv[...])
        botrow = jnp.where(my_x == 0, row_recv[...], x_ref[:, hh - 1:hh])

        rows_l = jnp.where(my_y == 0, own_l, col_recv[...])
        rows_r = jnp.where(my_y == 0, col_recv[...], own_r)
        inner_l = jnp.where(
            my_y == 0, row_recv[:, 0, :, 0], cor_recv[:, 0, :]
        )
        inner_r = jnp.where(
            my_y == 0, cor_recv[:, 0, :], row_recv[:, 0, :, ww - 1]
        )
        top_l = jnp.where(my_x == 0, rows_l[:, 0], inner_l)
        bot_l = jnp.where(my_x == 0, inner_l, rows_l[:, hh - 1])
        top_r = jnp.where(my_x == 0, rows_r[:, 0], inner_r)
        bot_r = jnp.where(my_x == 0, inner_r, rows_r[:, hh - 1])
        lcol = jnp.concatenate(
            [top_l[:, None], rows_l, bot_l[:, None]], axis=1
        )
        rcol = jnp.concatenate(
            [top_r[:, None], rows_r, bot_r[:, None]], axis=1
        )

        for pid in (nbx * 2 + my_y, my_x * 2 + nby, nbx * 2 + nby):
            rr = pltpu.make_async_remote_copy(
                src_ref=stat_send,
                dst_ref=stat_recv.at[pid],
                send_sem=send_sems.at[3],
                recv_sem=stat_sems.at[pid],
                device_id=(my_x, my_y), device_id_type=pl.DeviceIdType.MESH,
            )
            rr.wait_recv()

        tot = (
            stat_recv[0] + stat_recv[1] + stat_recv[2] + stat_recv[3]
        )
        mean = tot[0] / n_global
        var = tot[1] / n_global - mean * mean
        inv = lax.rsqrt(var + eps)

        kk = k_ref[...]
        ksum = jnp.sum(kk, axis=(0, 1))
        CH = 64
        bf = jnp.bfloat16
        wpb = jnp.broadcast_to(wp_ref[...].astype(bf), (CH, c, c))
        for bi in range(b):
            kb = (kk * inv[bi]).astype(bf)
            off = (inv[bi] * mean[bi] * ksum).astype(bf).reshape(1, c, 1)
            for h0 in range(0, hh, CH):
                pieces = []
                if h0 == 0:
                    pieces.append(toprow[bi])
                    lo = 0
                else:
                    lo = h0 - 1
                hi = min(h0 + CH + 1, hh)
                pieces.append(x_ref[bi, lo:hi])
                if hi < h0 + CH + 1:
                    pieces.append(botrow[bi])
                u1 = (
                    jnp.concatenate(pieces, axis=0)
                    if len(pieces) > 1 else pieces[0]
                ).astype(bf)
                lch = lcol[bi, h0:h0 + CH + 2].astype(bf)
                rch = rcol[bi, h0:h0 + CH + 2].astype(bf)
                u0 = jnp.concatenate([lch[:, :, None], u1[:, :, :ww - 1]], axis=2)
                u2 = jnp.concatenate([u1[:, :, 1:], rch[:, :, None]], axis=2)
                conv = jnp.zeros((CH, c, ww), bf) - off
                for dj, u in ((0, u0), (1, u1), (2, u2)):
                    for di in range(3):
                        conv = conv + u[di:di + CH] * kb[di, dj][:, None]
                a = conv * jax.nn.sigmoid(conv)
                y = lax.dot_general(
                    wpb, a, (((1,), (1,)), ((0,), (0,))),
                    preferred_element_type=jnp.float32,
                )
                out_ref[bi, h0:h0 + CH] = x_ref[bi, h0:h0 + CH] + y

        rdma_row.wait_send()
        rdma_col.wait_send()
        rdma_cor.wait_send()
        for r in stat_rdmas:
            r.wait_send()

    xt = jnp.transpose(x, (0, 1, 3, 2))
    out_t = pl.pallas_call(
        body,
        out_shape=jax.ShapeDtypeStruct((b, hh, c, ww), jnp.float32),
        in_specs=[
            pl.BlockSpec(memory_space=pltpu.VMEM),
            pl.BlockSpec(memory_space=pltpu.VMEM),
            pl.BlockSpec(memory_space=pltpu.VMEM),
        ],
        out_specs=pl.BlockSpec(memory_space=pltpu.VMEM),
        scratch_shapes=[
            pltpu.VMEM((b, 1, c, ww), jnp.float32),
            pltpu.VMEM((b, hh, c), jnp.float32),
            pltpu.VMEM((b, 1, c), jnp.float32),
            pltpu.VMEM((2, b, c), jnp.float32),
            pltpu.VMEM((b, 1, c, ww), jnp.float32),
            pltpu.VMEM((b, hh, c), jnp.float32),
            pltpu.VMEM((b, 1, c), jnp.float32),
            pltpu.VMEM((4, 2, b, c), jnp.float32),
            pltpu.SemaphoreType.DMA((6,)),
            pltpu.SemaphoreType.DMA((3,)),
            pltpu.SemaphoreType.DMA((4,)),
        ],
        compiler_params=pltpu.CompilerParams(
            collective_id=0, vmem_limit_bytes=120 * 1024 * 1024
        ),
    )(xt, k, Wp)
    return jnp.transpose(out_t, (0, 1, 3, 2))
